# Initial kernel scaffold; baseline (speedup 1.0000x reference)
import jax
import jax.numpy as jnp
from jax.experimental import pallas as pl


def kernel(x, adj0_index, adj1_index, params):
    raise NotImplementedError("write your pallas kernel here")


# speedup vs baseline: 4.5252x; 4.5252x over previous
import functools

import jax
import jax.numpy as jnp
from jax import lax
from jax.experimental import pallas as pl
from jax.experimental.pallas import tpu as pltpu
from jax.experimental.pallas import tpu_sc as plsc

N = 10000
E = 320000
FEAT = 128
NOUT = 40
NC = 2
NS = 16
NW = NC * NS
EPW = E // NW
KE = 80
NCHUNK = EPW // KE
RPT = 632
NPAD = NS * RPT
DEGW = 128

_f32 = jnp.float32


@functools.lru_cache(maxsize=None)
def _spmm_fn(F):
  mesh = plsc.VectorSubcoreMesh(core_axis_name="c", subcore_axis_name="s")

  @functools.partial(
      pl.kernel,
      out_type=jax.ShapeDtypeStruct((NC * NPAD, F), _f32),
      mesh=mesh,
      scratch_types=[
          pltpu.VMEM_SHARED((NPAD, F), _f32),
          pltpu.VMEM((KE,), jnp.int32),
          pltpu.VMEM((KE,), jnp.int32),
          pltpu.VMEM((KE, F), _f32),
          pltpu.SemaphoreType.DMA,
      ],
  )
  def spmm(x_hbm, row_hbm, col_hbm, zeros_hbm, out_hbm, acc, rowb, colb,
           gbuf, sem):
    cid = lax.axis_index("c")
    sid = lax.axis_index("s")
    wid = sid * NC + cid
    r0 = pl.multiple_of(sid * RPT, 8)
    pltpu.sync_copy(zeros_hbm, acc.at[pl.ds(r0, RPT)])
    plsc.subcore_barrier()
    base = wid * EPW

    def chunk(i, carry):
      off = pl.multiple_of(base + i * KE, 8)
      pltpu.sync_copy(col_hbm.at[pl.ds(off, KE)], colb)
      pltpu.sync_copy(row_hbm.at[pl.ds(off, KE)], rowb)
      pltpu.async_copy(x_hbm.at[colb], gbuf, sem).wait()
      pltpu.sync_copy(gbuf, acc.at[rowb], add=True)
      return carry

    lax.fori_loop(0, NCHUNK, chunk, 0)
    plsc.subcore_barrier()
    pltpu.sync_copy(acc.at[pl.ds(r0, RPT)],
                    out_hbm.at[pl.ds(pl.multiple_of(cid * NPAD + r0, 8), RPT)])

  return spmm


def _spmm(xf, row, col, zeros128):
  out = _spmm_fn(FEAT)(xf, row, col, zeros128).reshape(NC, NPAD, FEAT)
  return out[:, :N]


@functools.lru_cache(maxsize=None)
def _deg_fn():
  mesh = plsc.VectorSubcoreMesh(core_axis_name="c", subcore_axis_name="s")

  @functools.partial(
      pl.kernel,
      out_type=jax.ShapeDtypeStruct((NC * NPAD, DEGW), _f32),
      mesh=mesh,
      scratch_types=[
          pltpu.VMEM_SHARED((NPAD, DEGW), _f32),
          pltpu.VMEM((KE,), jnp.int32),
          pltpu.VMEM((KE, DEGW), _f32),
      ],
  )
  def deg(row_hbm, ones_hbm, zeros_hbm, out_hbm, acc, rowb, onesb):
    cid = lax.axis_index("c")
    sid = lax.axis_index("s")
    wid = sid * NC + cid
    r0 = pl.multiple_of(sid * RPT, 8)
    pltpu.sync_copy(zeros_hbm, acc.at[pl.ds(r0, RPT)])
    pltpu.sync_copy(ones_hbm, onesb)
    plsc.subcore_barrier()
    base = wid * EPW

    def chunk(i, carry):
      off = pl.multiple_of(base + i * KE, 8)
      pltpu.sync_copy(row_hbm.at[pl.ds(off, KE)], rowb)
      pltpu.sync_copy(onesb, acc.at[rowb], add=True)
      return carry

    lax.fori_loop(0, NCHUNK, chunk, 0)
    plsc.subcore_barrier()
    pltpu.sync_copy(acc.at[pl.ds(r0, RPT)],
                    out_hbm.at[pl.ds(pl.multiple_of(cid * NPAD + r0, 8), RPT)])

  return deg


def _deg(row, onesw, zerosw):
  out = _deg_fn()(row, onesw, zerosw).reshape(NC, NPAD, DEGW)
  return out[:, :N]


def _dot(a, b):
  return jnp.dot(a, b, preferred_element_type=_f32,
                 precision=lax.Precision.HIGHEST)


def _dinv_of(d_block):
  deg = d_block[0, :, 0:1] + d_block[1, :, 0:1]
  return lax.rsqrt(jnp.maximum(deg, 1e-6))


def _bn_elu_expr(z, gamma, beta):
  m = jnp.mean(z, axis=0, keepdims=True)
  v = jnp.mean((z - m) ** 2, axis=0, keepdims=True)
  zn = (z - m) * lax.rsqrt(v + 1e-5) * gamma + beta
  return jnp.where(zn > 0, zn, jnp.exp(jnp.minimum(zn, 0.0)) - 1.0)


def _prep(x_in, degp, w_gcn):
  def body(x_ref, d_ref, w_ref, sup_ref, y_ref):
    x_ = x_ref[...]
    sup_ref[...] = _dot(x_, w_ref[...])
    y_ref[...] = _dinv_of(d_ref) * x_

  sh = jax.ShapeDtypeStruct((N, FEAT), _f32)
  return pl.pallas_call(body, out_shape=(sh, sh))(x_in, degp, w_gcn)


def _mid(up, degp):
  def body(up_ref, d_ref, t1_ref, w2_ref):
    dinv = _dinv_of(d_ref)
    t1 = -dinv * (up_ref[0] + up_ref[1])
    t1_ref[...] = t1
    w2_ref[...] = dinv * t1

  sh = jax.ShapeDtypeStruct((N, FEAT), _f32)
  return pl.pallas_call(body, out_shape=(sh, sh))(up, degp)


_BLK = 2000


def _conv_pre(gp, vp, x0, t1, degp, cheb_w, bias, a):
  def body(gp_ref, vp_ref, x0_ref, t1_ref, d_ref, w_ref, b_ref, a_ref,
           out_ref):
    dinv = _dinv_of(d_ref)
    out_gcn = gp_ref[0] + gp_ref[1]
    x0_ = x0_ref[...]
    t1_ = t1_ref[...]
    t2 = -2.0 * dinv * (vp_ref[0] + vp_ref[1]) - x0_
    w0 = w_ref[0]
    w1 = w_ref[1]
    w2 = w_ref[2]
    n0 = jnp.sqrt(jnp.sum(w0 * w0))
    n1 = jnp.sqrt(jnp.sum(w1 * w1))
    n2 = jnp.sqrt(jnp.sum(w2 * w2))
    cheb = (_dot(x0_, w0 + 0.01 * n0) + _dot(t1_, w1 + 0.01 * n1)
            + _dot(t2, w2 + 0.01 * n2)) * 0.001
    a_ = a_ref[0, 0]
    out_ref[...] = a_ * out_gcn + (1.0 - a_) * cheb + b_ref[...]

  return pl.pallas_call(
      body,
      grid=(N // _BLK,),
      in_specs=[
          pl.BlockSpec((2, _BLK, FEAT), lambda i: (0, i, 0)),
          pl.BlockSpec((2, _BLK, FEAT), lambda i: (0, i, 0)),
          pl.BlockSpec((_BLK, FEAT), lambda i: (i, 0)),
          pl.BlockSpec((_BLK, FEAT), lambda i: (i, 0)),
          pl.BlockSpec((2, _BLK, DEGW), lambda i: (0, i, 0)),
          pl.BlockSpec((3, FEAT, FEAT), lambda i: (0, 0, 0)),
          pl.BlockSpec((1, FEAT), lambda i: (0, 0)),
          pl.BlockSpec(memory_space=pltpu.SMEM),
      ],
      out_specs=pl.BlockSpec((_BLK, FEAT), lambda i: (i, 0)),
      out_shape=jax.ShapeDtypeStruct((N, FEAT), _f32),
  )(gp, vp, x0, t1, degp, cheb_w, bias, a)


def _bn_elu(z, gamma, beta):
  def body(z_ref, g_ref, b_ref, out_ref):
    out_ref[...] = _bn_elu_expr(z_ref[...], g_ref[...], b_ref[...])

  return pl.pallas_call(
      body, out_shape=jax.ShapeDtypeStruct((N, FEAT), _f32))(z, gamma, beta)


def _head(h0, h1, w11, b11, g1, be1, w111, b111, g2, be2, w12, b12):
  def body(h0_ref, h1_ref, w11_ref, b11_ref, g1_ref, be1_ref, w111_ref,
           b111_ref, g2_ref, be2_ref, w12_ref, b12_ref, out_ref):
    x1 = jnp.concatenate([h0_ref[...], h1_ref[...]], axis=1)
    z = _dot(x1, w11_ref[...]) + b11_ref[...]
    z = _bn_elu_expr(z, g1_ref[...], be1_ref[...])
    z = _dot(z, w111_ref[...]) + b111_ref[...]
    z = _bn_elu_expr(z, g2_ref[...], be2_ref[...])
    z = _dot(z, w12_ref[...]) + b12_ref[...]
    m = jnp.max(z, axis=1, keepdims=True)
    s = z - m
    lse = jnp.log(jnp.sum(jnp.exp(s), axis=1, keepdims=True))
    out_ref[...] = s - lse

  return pl.pallas_call(
      body, out_shape=jax.ShapeDtypeStruct((N, NOUT), _f32))(
          h0, h1, w11, b11, g1, be1, w111, b111, g2, be2, w12, b12)


def _conv(x_in, row, col, degp, p, bnp, zeros128):
  a = jax.nn.sigmoid(p['alpha']).reshape(1, 1)
  sup, y = _prep(x_in, degp, p['gcn_weight'])
  g = _spmm(sup, row, col, zeros128)
  u = _spmm(y, row, col, zeros128)
  t1, w2 = _mid(u, degp)
  v = _spmm(w2, row, col, zeros128)
  pre = _conv_pre(g, v, x_in, t1, degp, p['cheb_weight'],
                  p['bias'].reshape(1, FEAT), a)
  return _bn_elu(pre, bnp['gamma'].reshape(1, FEAT),
                 bnp['beta'].reshape(1, FEAT))


def kernel(x, adj0_index, adj1_index, params):
  row0, col0 = adj0_index[0], adj0_index[1]
  row1, col1 = adj1_index[0], adj1_index[1]
  zeros128 = jnp.zeros((RPT, FEAT), _f32)
  zerosw = jnp.zeros((RPT, DEGW), _f32)
  onesw = jnp.ones((KE, DEGW), _f32)

  degp0 = _deg(row0, onesw, zerosw)
  degp1 = _deg(row1, onesw, zerosw)

  h_e = _conv(x, row0, col0, degp0, params['gc_in_exp'],
              params['bn_in_exp'], zeros128)
  h_s = _conv(x, row1, col1, degp1, params['gc_in_sp'],
              params['bn_in_sp'], zeros128)
  h_e = _conv(h_e, row0, col0, degp0, params['cgc1_exp'],
              params['bn_c1_exp'], zeros128)
  h_s = _conv(h_s, row1, col1, degp1, params['cgc1_sp'],
              params['bn_c1_sp'], zeros128)

  return _head(
      h_e, h_s,
      params['W_out11'], params['b_out11'].reshape(1, FEAT),
      params['bn_out1']['gamma'].reshape(1, FEAT),
      params['bn_out1']['beta'].reshape(1, FEAT),
      params['W_out111'], params['b_out111'].reshape(1, FEAT),
      params['bn_out111']['gamma'].reshape(1, FEAT),
      params['bn_out111']['beta'].reshape(1, FEAT),
      params['W_out12'], params['b_out12'].reshape(1, NOUT))


# speedup vs baseline: 5.7402x; 1.2685x over previous
import functools

import jax
import jax.numpy as jnp
from jax import lax
from jax.experimental import pallas as pl
from jax.experimental.pallas import tpu as pltpu
from jax.experimental.pallas import tpu_sc as plsc

N = 10000
E = 320000
FEAT = 128
NOUT = 40
NC = 2
NS = 16
NW = NC * NS
EPW = E // NW
KE = 80
NCHUNK = 126
EPWP = NCHUNK * KE
NSLOT = 3
NGROUPS = NCHUNK // NSLOT
RPT = 632
NPAD = NS * RPT
DEGW = 128

_f32 = jnp.float32


@functools.lru_cache(maxsize=None)
def _spmm_fn(F):
  mesh = plsc.VectorSubcoreMesh(core_axis_name="c", subcore_axis_name="s")

  scratch = ([pltpu.VMEM_SHARED((NPAD, F), _f32)]
             + [pltpu.VMEM((KE,), jnp.int32) for _ in range(2 * NSLOT)]
             + [pltpu.VMEM((KE, F), _f32) for _ in range(NSLOT)]
             + [pltpu.SemaphoreType.DMA for _ in range(2 * NSLOT)])

  @functools.partial(
      pl.kernel,
      out_type=jax.ShapeDtypeStruct((NC * NPAD, F), _f32),
      mesh=mesh,
      scratch_types=scratch,
  )
  def spmm(x_hbm, row_hbm, col_hbm, zeros_hbm, out_hbm, acc, *rest):
    rowbs = rest[0:NSLOT]
    colbs = rest[NSLOT:2 * NSLOT]
    gbufs = rest[2 * NSLOT:3 * NSLOT]
    gsems = rest[3 * NSLOT:4 * NSLOT]
    ssems = rest[4 * NSLOT:5 * NSLOT]
    cid = lax.axis_index("c")
    sid = lax.axis_index("s")
    wid = sid * NC + cid
    r0 = pl.multiple_of(sid * RPT, 8)
    pltpu.sync_copy(zeros_hbm, acc.at[pl.ds(r0, RPT)])
    plsc.subcore_barrier()

    def stage_and_gather(c, b):
      pltpu.sync_copy(row_hbm.at[wid, c], rowbs[b])
      pltpu.sync_copy(col_hbm.at[wid, c], colbs[b])
      pltpu.async_copy(x_hbm.at[colbs[b]], gbufs[b], gsems[b])

    def wait_g_fire_s(b):
      pltpu.make_async_copy(x_hbm.at[colbs[b]], gbufs[b], gsems[b]).wait()
      pltpu.async_copy(gbufs[b], acc.at[rowbs[b]], ssems[b], add=True)

    def wait_s(b):
      pltpu.make_async_copy(gbufs[b], acc.at[rowbs[b]], ssems[b]).wait()

    def group(m, carry):
      for b in range(NSLOT):
        c = m * NSLOT + b
        prev = (b + NSLOT - 1) % NSLOT

        @pl.when(m >= 1)
        def _():
          wait_s(b)

        stage_and_gather(c, b)
        if b == 0:
          @pl.when(m >= 1)
          def _():
            wait_g_fire_s(prev)
        else:
          wait_g_fire_s(prev)
      return carry

    lax.fori_loop(0, NGROUPS, group, 0)
    wait_g_fire_s(NSLOT - 1)
    for b in range(NSLOT):
      wait_s(b)
    plsc.subcore_barrier()
    pltpu.sync_copy(acc.at[pl.ds(r0, RPT)],
                    out_hbm.at[pl.ds(pl.multiple_of(cid * NPAD + r0, 8), RPT)])

  return spmm


@functools.lru_cache(maxsize=None)
def _deg_fn():
  mesh = plsc.VectorSubcoreMesh(core_axis_name="c", subcore_axis_name="s")

  scratch = ([
      pltpu.VMEM_SHARED((NPAD, DEGW), _f32),
      pltpu.VMEM((NCHUNK, KE), jnp.int32),
      pltpu.VMEM((KE, DEGW), _f32),
  ] + [pltpu.SemaphoreType.DMA for _ in range(NSLOT)])

  @functools.partial(
      pl.kernel,
      out_type=jax.ShapeDtypeStruct((NC * NPAD, DEGW), _f32),
      mesh=mesh,
      scratch_types=scratch,
  )
  def deg(row_hbm, ones_hbm, zeros_hbm, out_hbm, acc, rowb, onesb, *ssems):
    cid = lax.axis_index("c")
    sid = lax.axis_index("s")
    wid = sid * NC + cid
    r0 = pl.multiple_of(sid * RPT, 8)
    pltpu.sync_copy(zeros_hbm, acc.at[pl.ds(r0, RPT)])
    pltpu.sync_copy(row_hbm.at[wid], rowb)
    pltpu.sync_copy(ones_hbm, onesb)
    plsc.subcore_barrier()

    for b in range(NSLOT):
      pltpu.async_copy(onesb, acc.at[rowb.at[b]], ssems[b], add=True)

    def group(m, carry):
      for b in range(NSLOT):
        c = m * NSLOT + b
        pltpu.make_async_copy(onesb, acc.at[rowb.at[c]], ssems[b]).wait()
        pltpu.async_copy(onesb, acc.at[rowb.at[c]], ssems[b], add=True)
      return carry

    lax.fori_loop(1, NGROUPS, group, 0)
    for b in range(NSLOT):
      pltpu.make_async_copy(onesb, acc.at[rowb.at[b]], ssems[b]).wait()
    plsc.subcore_barrier()
    pltpu.sync_copy(acc.at[pl.ds(r0, RPT)],
                    out_hbm.at[pl.ds(pl.multiple_of(cid * NPAD + r0, 8), RPT)])

  return deg


def _deg(row, onesw, zerosw):
  out = _deg_fn()(row, onesw, zerosw).reshape(NC, NPAD, DEGW)
  return out[:, :N]


def _spmm(xf, row, col, zeros128):
  out = _spmm_fn(FEAT)(xf, row, col, zeros128).reshape(NC, NPAD, FEAT)
  return out[:, :N]


def _chunked(idx, fill):
  a = idx.reshape(NW, EPW)
  pad = jnp.full((NW, EPWP - EPW), fill, jnp.int32)
  return jnp.concatenate([a, pad], axis=1).reshape(NW, NCHUNK, KE)


def _dot(a, b):
  return jnp.dot(a, b, preferred_element_type=_f32,
                 precision=lax.Precision.HIGHEST)


def _dinv_of(d_block):
  deg = d_block[0, :, 0:1] + d_block[1, :, 0:1]
  return lax.rsqrt(jnp.maximum(deg, 1e-6))


def _bn_elu_expr(z, gamma, beta):
  m = jnp.mean(z, axis=0, keepdims=True)
  v = jnp.mean((z - m) ** 2, axis=0, keepdims=True)
  zn = (z - m) * lax.rsqrt(v + 1e-5) * gamma + beta
  return jnp.where(zn > 0, zn, jnp.exp(jnp.minimum(zn, 0.0)) - 1.0)


def _prep(x_in, degp, w_gcn):
  def body(x_ref, d_ref, w_ref, sup_ref, y_ref):
    x_ = x_ref[...]
    sup_ref[...] = _dot(x_, w_ref[...])
    y_ref[...] = _dinv_of(d_ref) * x_

  sh = jax.ShapeDtypeStruct((N, FEAT), _f32)
  return pl.pallas_call(body, out_shape=(sh, sh))(x_in, degp, w_gcn)


def _mid(up, degp):
  def body(up_ref, d_ref, t1_ref, w2_ref):
    dinv = _dinv_of(d_ref)
    t1 = -dinv * (up_ref[0] + up_ref[1])
    t1_ref[...] = t1
    w2_ref[...] = dinv * t1

  sh = jax.ShapeDtypeStruct((N, FEAT), _f32)
  return pl.pallas_call(body, out_shape=(sh, sh))(up, degp)


_BLK = 2000


def _conv_pre(gp, vp, x0, t1, degp, cheb_w, bias, a):
  def body(gp_ref, vp_ref, x0_ref, t1_ref, d_ref, w_ref, b_ref, a_ref,
           out_ref):
    dinv = _dinv_of(d_ref)
    out_gcn = gp_ref[0] + gp_ref[1]
    x0_ = x0_ref[...]
    t1_ = t1_ref[...]
    t2 = -2.0 * dinv * (vp_ref[0] + vp_ref[1]) - x0_
    w0 = w_ref[0]
    w1 = w_ref[1]
    w2 = w_ref[2]
    n0 = jnp.sqrt(jnp.sum(w0 * w0))
    n1 = jnp.sqrt(jnp.sum(w1 * w1))
    n2 = jnp.sqrt(jnp.sum(w2 * w2))
    cheb = (_dot(x0_, w0 + 0.01 * n0) + _dot(t1_, w1 + 0.01 * n1)
            + _dot(t2, w2 + 0.01 * n2)) * 0.001
    a_ = a_ref[0, 0]
    out_ref[...] = a_ * out_gcn + (1.0 - a_) * cheb + b_ref[...]

  return pl.pallas_call(
      body,
      grid=(N // _BLK,),
      in_specs=[
          pl.BlockSpec((2, _BLK, FEAT), lambda i: (0, i, 0)),
          pl.BlockSpec((2, _BLK, FEAT), lambda i: (0, i, 0)),
          pl.BlockSpec((_BLK, FEAT), lambda i: (i, 0)),
          pl.BlockSpec((_BLK, FEAT), lambda i: (i, 0)),
          pl.BlockSpec((2, _BLK, DEGW), lambda i: (0, i, 0)),
          pl.BlockSpec((3, FEAT, FEAT), lambda i: (0, 0, 0)),
          pl.BlockSpec((1, FEAT), lambda i: (0, 0)),
          pl.BlockSpec(memory_space=pltpu.SMEM),
      ],
      out_specs=pl.BlockSpec((_BLK, FEAT), lambda i: (i, 0)),
      out_shape=jax.ShapeDtypeStruct((N, FEAT), _f32),
  )(gp, vp, x0, t1, degp, cheb_w, bias, a)


def _bn_elu(z, gamma, beta):
  def body(z_ref, g_ref, b_ref, out_ref):
    out_ref[...] = _bn_elu_expr(z_ref[...], g_ref[...], b_ref[...])

  return pl.pallas_call(
      body, out_shape=jax.ShapeDtypeStruct((N, FEAT), _f32))(z, gamma, beta)


def _head(h0, h1, w11, b11, g1, be1, w111, b111, g2, be2, w12, b12):
  def body(h0_ref, h1_ref, w11_ref, b11_ref, g1_ref, be1_ref, w111_ref,
           b111_ref, g2_ref, be2_ref, w12_ref, b12_ref, out_ref):
    x1 = jnp.concatenate([h0_ref[...], h1_ref[...]], axis=1)
    z = _dot(x1, w11_ref[...]) + b11_ref[...]
    z = _bn_elu_expr(z, g1_ref[...], be1_ref[...])
    z = _dot(z, w111_ref[...]) + b111_ref[...]
    z = _bn_elu_expr(z, g2_ref[...], be2_ref[...])
    z = _dot(z, w12_ref[...]) + b12_ref[...]
    m = jnp.max(z, axis=1, keepdims=True)
    s = z - m
    lse = jnp.log(jnp.sum(jnp.exp(s), axis=1, keepdims=True))
    out_ref[...] = s - lse

  return pl.pallas_call(
      body, out_shape=jax.ShapeDtypeStruct((N, NOUT), _f32))(
          h0, h1, w11, b11, g1, be1, w111, b111, g2, be2, w12, b12)


def _conv(x_in, row, col, degp, p, bnp, zeros128):
  a = jax.nn.sigmoid(p['alpha']).reshape(1, 1)
  sup, y = _prep(x_in, degp, p['gcn_weight'])
  g = _spmm(sup, row, col, zeros128)
  u = _spmm(y, row, col, zeros128)
  t1, w2 = _mid(u, degp)
  v = _spmm(w2, row, col, zeros128)
  pre = _conv_pre(g, v, x_in, t1, degp, p['cheb_weight'],
                  p['bias'].reshape(1, FEAT), a)
  return _bn_elu(pre, bnp['gamma'].reshape(1, FEAT),
                 bnp['beta'].reshape(1, FEAT))


def kernel(x, adj0_index, adj1_index, params):
  row0, col0 = _chunked(adj0_index[0], N), _chunked(adj0_index[1], 0)
  row1, col1 = _chunked(adj1_index[0], N), _chunked(adj1_index[1], 0)
  zeros128 = jnp.zeros((RPT, FEAT), _f32)
  zerosw = jnp.zeros((RPT, DEGW), _f32)
  onesw = jnp.ones((KE, DEGW), _f32)

  degp0 = _deg(row0, onesw, zerosw)
  degp1 = _deg(row1, onesw, zerosw)

  h_e = _conv(x, row0, col0, degp0, params['gc_in_exp'],
              params['bn_in_exp'], zeros128)
  h_s = _conv(x, row1, col1, degp1, params['gc_in_sp'],
              params['bn_in_sp'], zeros128)
  h_e = _conv(h_e, row0, col0, degp0, params['cgc1_exp'],
              params['bn_c1_exp'], zeros128)
  h_s = _conv(h_s, row1, col1, degp1, params['cgc1_sp'],
              params['bn_c1_sp'], zeros128)

  return _head(
      h_e, h_s,
      params['W_out11'], params['b_out11'].reshape(1, FEAT),
      params['bn_out1']['gamma'].reshape(1, FEAT),
      params['bn_out1']['beta'].reshape(1, FEAT),
      params['W_out111'], params['b_out111'].reshape(1, FEAT),
      params['bn_out111']['gamma'].reshape(1, FEAT),
      params['bn_out111']['beta'].reshape(1, FEAT),
      params['W_out12'], params['b_out12'].reshape(1, NOUT))


# speedup vs baseline: 6.9579x; 1.2121x over previous
import functools

import jax
import jax.numpy as jnp
from jax import lax
from jax.experimental import pallas as pl
from jax.experimental.pallas import tpu as pltpu
from jax.experimental.pallas import tpu_sc as plsc

N = 10000
E = 320000
FEAT = 128
NOUT = 40
NC = 2
NS = 16
NW = NC * NS
EPW = E // NW
KE = 112
NCHUNK = 90
EPWP = NCHUNK * KE
NSLOT = 3
NGROUPS = NCHUNK // NSLOT
RPT = 632
NPAD = NS * RPT
DEGW = 128

_f32 = jnp.float32


@functools.lru_cache(maxsize=None)
def _spmm_fn(F):
  mesh = plsc.VectorSubcoreMesh(core_axis_name="c", subcore_axis_name="s")

  scratch = ([pltpu.VMEM_SHARED((NPAD, F), _f32)]
             + [pltpu.VMEM((KE,), jnp.int32) for _ in range(4 * NSLOT)]
             + [pltpu.VMEM((KE, F), _f32) for _ in range(NSLOT)]
             + [pltpu.SemaphoreType.DMA for _ in range(3 * NSLOT)])

  @functools.partial(
      pl.kernel,
      out_type=jax.ShapeDtypeStruct((NC * NPAD, F), _f32),
      mesh=mesh,
      scratch_types=scratch,
  )
  def spmm(x_hbm, row_hbm, col_hbm, zeros_hbm, out_hbm, acc, *rest):
    rowbs = [rest[2 * b:2 * b + 2] for b in range(NSLOT)]
    colbs = [rest[2 * NSLOT + 2 * b:2 * NSLOT + 2 * b + 2]
             for b in range(NSLOT)]
    gbufs = rest[4 * NSLOT:5 * NSLOT]
    gsems = rest[5 * NSLOT:6 * NSLOT]
    ssems = rest[6 * NSLOT:7 * NSLOT]
    isems = rest[7 * NSLOT:8 * NSLOT]
    cid = lax.axis_index("c")
    sid = lax.axis_index("s")
    wid = sid * NC + cid
    r0 = pl.multiple_of(sid * RPT, 8)
    pltpu.sync_copy(zeros_hbm, acc.at[pl.ds(r0, RPT)])
    plsc.subcore_barrier()

    def fire_idx(c, b, p):
      pltpu.async_copy(row_hbm.at[wid, c], rowbs[b][p], isems[b])
      pltpu.async_copy(col_hbm.at[wid, c], colbs[b][p], isems[b])

    def wait_idx(b, p):
      pltpu.make_async_copy(row_hbm.at[wid, 0], rowbs[b][p],
                            isems[b]).wait()
      pltpu.make_async_copy(col_hbm.at[wid, 0], colbs[b][p],
                            isems[b]).wait()

    def fire_g(b, p):
      pltpu.async_copy(x_hbm.at[colbs[b][p]], gbufs[b], gsems[b])

    def wait_g_fire_s(b, p):
      pltpu.make_async_copy(x_hbm.at[colbs[b][p]], gbufs[b],
                            gsems[b]).wait()
      pltpu.async_copy(gbufs[b], acc.at[rowbs[b][p]], ssems[b], add=True)

    def wait_s(b, p):
      pltpu.make_async_copy(gbufs[b], acc.at[rowbs[b][p]], ssems[b]).wait()

    for b in range(NSLOT):
      fire_idx(b, b, 0)

    def group_pair(m2, carry):
      for p in (0, 1):
        m = 2 * m2 + p
        for b in range(NSLOT):
          c = m * NSLOT + b
          prev = (b + NSLOT - 1) % NSLOT
          if p == 0:
            @pl.when(m >= 1)
            def _():
              wait_s(b, p)
          else:
            wait_s(b, p)
          wait_idx(b, p)
          fire_g(b, p)

          @pl.when(m < NGROUPS - 1)
          def _():
            fire_idx(c + NSLOT, b, 1 - p)

          if b == 0:
            if p == 0:
              @pl.when(m >= 1)
              def _():
                wait_g_fire_s(prev, 1 - p)
            else:
              wait_g_fire_s(prev, 1 - p)
          else:
            wait_g_fire_s(prev, p)
      return carry

    lax.fori_loop(0, NGROUPS // 2, group_pair, 0)
    wait_g_fire_s(NSLOT - 1, (NGROUPS - 1) % 2)
    for b in range(NSLOT):
      wait_s(b, 0)
    plsc.subcore_barrier()
    pltpu.sync_copy(acc.at[pl.ds(r0, RPT)],
                    out_hbm.at[pl.ds(pl.multiple_of(cid * NPAD + r0, 8), RPT)])

  return spmm


@functools.lru_cache(maxsize=None)
def _deg_fn():
  mesh = plsc.VectorSubcoreMesh(core_axis_name="c", subcore_axis_name="s")

  scratch = ([
      pltpu.VMEM_SHARED((NPAD, DEGW), _f32),
      pltpu.VMEM((NCHUNK, KE), jnp.int32),
      pltpu.VMEM((KE, DEGW), _f32),
  ] + [pltpu.SemaphoreType.DMA for _ in range(NSLOT)])

  @functools.partial(
      pl.kernel,
      out_type=jax.ShapeDtypeStruct((NC * NPAD, DEGW), _f32),
      mesh=mesh,
      scratch_types=scratch,
  )
  def deg(row_hbm, ones_hbm, zeros_hbm, out_hbm, acc, rowb, onesb, *ssems):
    cid = lax.axis_index("c")
    sid = lax.axis_index("s")
    wid = sid * NC + cid
    r0 = pl.multiple_of(sid * RPT, 8)
    pltpu.sync_copy(zeros_hbm, acc.at[pl.ds(r0, RPT)])
    pltpu.sync_copy(row_hbm.at[wid], rowb)
    pltpu.sync_copy(ones_hbm, onesb)
    plsc.subcore_barrier()

    for b in range(NSLOT):
      pltpu.async_copy(onesb, acc.at[rowb.at[b]], ssems[b], add=True)

    def group(m, carry):
      for b in range(NSLOT):
        c = m * NSLOT + b
        pltpu.make_async_copy(onesb, acc.at[rowb.at[c]], ssems[b]).wait()
        pltpu.async_copy(onesb, acc.at[rowb.at[c]], ssems[b], add=True)
      return carry

    lax.fori_loop(1, NGROUPS, group, 0)
    for b in range(NSLOT):
      pltpu.make_async_copy(onesb, acc.at[rowb.at[b]], ssems[b]).wait()
    plsc.subcore_barrier()
    pltpu.sync_copy(acc.at[pl.ds(r0, RPT)],
                    out_hbm.at[pl.ds(pl.multiple_of(cid * NPAD + r0, 8), RPT)])

  return deg


def _deg(row, onesw, zerosw):
  out = _deg_fn()(row, onesw, zerosw).reshape(NC, NPAD, DEGW)
  return out[:, :N]


def _spmm(xf, row, col, zeros128):
  out = _spmm_fn(FEAT)(xf, row, col, zeros128).reshape(NC, NPAD, FEAT)
  return out[:, :N]


def _chunked(idx, fill):
  a = idx.reshape(NW, EPW)
  pad = jnp.full((NW, EPWP - EPW), fill, jnp.int32)
  return jnp.concatenate([a, pad], axis=1).reshape(NW, NCHUNK, KE)


def _dot(a, b):
  return jnp.dot(a, b, preferred_element_type=_f32,
                 precision=lax.Precision.HIGHEST)


def _dinv_of(d_block):
  deg = d_block[0, :, 0:1] + d_block[1, :, 0:1]
  return lax.rsqrt(jnp.maximum(deg, 1e-6))


def _bn_elu_expr(z, gamma, beta):
  m = jnp.mean(z, axis=0, keepdims=True)
  v = jnp.mean((z - m) ** 2, axis=0, keepdims=True)
  zn = (z - m) * lax.rsqrt(v + 1e-5) * gamma + beta
  return jnp.where(zn > 0, zn, jnp.exp(jnp.minimum(zn, 0.0)) - 1.0)


def _prep(x_in, degp, w_gcn):
  def body(x_ref, d_ref, w_ref, sup_ref, y_ref):
    x_ = x_ref[...]
    sup_ref[...] = _dot(x_, w_ref[...])
    y_ref[...] = _dinv_of(d_ref) * x_

  sh = jax.ShapeDtypeStruct((N, FEAT), _f32)
  return pl.pallas_call(body, out_shape=(sh, sh))(x_in, degp, w_gcn)


def _mid(up, degp):
  def body(up_ref, d_ref, t1_ref, w2_ref):
    dinv = _dinv_of(d_ref)
    t1 = -dinv * (up_ref[0] + up_ref[1])
    t1_ref[...] = t1
    w2_ref[...] = dinv * t1

  sh = jax.ShapeDtypeStruct((N, FEAT), _f32)
  return pl.pallas_call(body, out_shape=(sh, sh))(up, degp)


_BLK = 2000


def _conv_pre(gp, vp, x0, t1, degp, cheb_w, bias, a):
  def body(gp_ref, vp_ref, x0_ref, t1_ref, d_ref, w_ref, b_ref, a_ref,
           out_ref):
    dinv = _dinv_of(d_ref)
    out_gcn = gp_ref[0] + gp_ref[1]
    x0_ = x0_ref[...]
    t1_ = t1_ref[...]
    t2 = -2.0 * dinv * (vp_ref[0] + vp_ref[1]) - x0_
    w0 = w_ref[0]
    w1 = w_ref[1]
    w2 = w_ref[2]
    n0 = jnp.sqrt(jnp.sum(w0 * w0))
    n1 = jnp.sqrt(jnp.sum(w1 * w1))
    n2 = jnp.sqrt(jnp.sum(w2 * w2))
    cheb = (_dot(x0_, w0 + 0.01 * n0) + _dot(t1_, w1 + 0.01 * n1)
            + _dot(t2, w2 + 0.01 * n2)) * 0.001
    a_ = a_ref[0, 0]
    out_ref[...] = a_ * out_gcn + (1.0 - a_) * cheb + b_ref[...]

  return pl.pallas_call(
      body,
      grid=(N // _BLK,),
      in_specs=[
          pl.BlockSpec((2, _BLK, FEAT), lambda i: (0, i, 0)),
          pl.BlockSpec((2, _BLK, FEAT), lambda i: (0, i, 0)),
          pl.BlockSpec((_BLK, FEAT), lambda i: (i, 0)),
          pl.BlockSpec((_BLK, FEAT), lambda i: (i, 0)),
          pl.BlockSpec((2, _BLK, DEGW), lambda i: (0, i, 0)),
          pl.BlockSpec((3, FEAT, FEAT), lambda i: (0, 0, 0)),
          pl.BlockSpec((1, FEAT), lambda i: (0, 0)),
          pl.BlockSpec(memory_space=pltpu.SMEM),
      ],
      out_specs=pl.BlockSpec((_BLK, FEAT), lambda i: (i, 0)),
      out_shape=jax.ShapeDtypeStruct((N, FEAT), _f32),
  )(gp, vp, x0, t1, degp, cheb_w, bias, a)


def _bn_elu(z, gamma, beta):
  def body(z_ref, g_ref, b_ref, out_ref):
    out_ref[...] = _bn_elu_expr(z_ref[...], g_ref[...], b_ref[...])

  return pl.pallas_call(
      body, out_shape=jax.ShapeDtypeStruct((N, FEAT), _f32))(z, gamma, beta)


def _head(h0, h1, w11, b11, g1, be1, w111, b111, g2, be2, w12, b12):
  def body(h0_ref, h1_ref, w11_ref, b11_ref, g1_ref, be1_ref, w111_ref,
           b111_ref, g2_ref, be2_ref, w12_ref, b12_ref, out_ref):
    x1 = jnp.concatenate([h0_ref[...], h1_ref[...]], axis=1)
    z = _dot(x1, w11_ref[...]) + b11_ref[...]
    z = _bn_elu_expr(z, g1_ref[...], be1_ref[...])
    z = _dot(z, w111_ref[...]) + b111_ref[...]
    z = _bn_elu_expr(z, g2_ref[...], be2_ref[...])
    z = _dot(z, w12_ref[...]) + b12_ref[...]
    m = jnp.max(z, axis=1, keepdims=True)
    s = z - m
    lse = jnp.log(jnp.sum(jnp.exp(s), axis=1, keepdims=True))
    out_ref[...] = s - lse

  return pl.pallas_call(
      body, out_shape=jax.ShapeDtypeStruct((N, NOUT), _f32))(
          h0, h1, w11, b11, g1, be1, w111, b111, g2, be2, w12, b12)


def _conv(x_in, row, col, degp, p, bnp, zeros128):
  a = jax.nn.sigmoid(p['alpha']).reshape(1, 1)
  sup, y = _prep(x_in, degp, p['gcn_weight'])
  g = _spmm(sup, row, col, zeros128)
  u = _spmm(y, row, col, zeros128)
  t1, w2 = _mid(u, degp)
  v = _spmm(w2, row, col, zeros128)
  pre = _conv_pre(g, v, x_in, t1, degp, p['cheb_weight'],
                  p['bias'].reshape(1, FEAT), a)
  return _bn_elu(pre, bnp['gamma'].reshape(1, FEAT),
                 bnp['beta'].reshape(1, FEAT))


def kernel(x, adj0_index, adj1_index, params):
  row0, col0 = _chunked(adj0_index[0], N), _chunked(adj0_index[1], 0)
  row1, col1 = _chunked(adj1_index[0], N), _chunked(adj1_index[1], 0)
  zeros128 = jnp.zeros((RPT, FEAT), _f32)
  zerosw = jnp.zeros((RPT, DEGW), _f32)
  onesw = jnp.ones((KE, DEGW), _f32)

  degp0 = _deg(row0, onesw, zerosw)
  degp1 = _deg(row1, onesw, zerosw)

  h_e = _conv(x, row0, col0, degp0, params['gc_in_exp'],
              params['bn_in_exp'], zeros128)
  h_s = _conv(x, row1, col1, degp1, params['gc_in_sp'],
              params['bn_in_sp'], zeros128)
  h_e = _conv(h_e, row0, col0, degp0, params['cgc1_exp'],
              params['bn_c1_exp'], zeros128)
  h_s = _conv(h_s, row1, col1, degp1, params['cgc1_sp'],
              params['bn_c1_sp'], zeros128)

  return _head(
      h_e, h_s,
      params['W_out11'], params['b_out11'].reshape(1, FEAT),
      params['bn_out1']['gamma'].reshape(1, FEAT),
      params['bn_out1']['beta'].reshape(1, FEAT),
      params['W_out111'], params['b_out111'].reshape(1, FEAT),
      params['bn_out111']['gamma'].reshape(1, FEAT),
      params['bn_out111']['beta'].reshape(1, FEAT),
      params['W_out12'], params['b_out12'].reshape(1, NOUT))


# speedup vs baseline: 9.2052x; 1.3230x over previous
import functools

import jax
import jax.numpy as jnp
from jax import lax
from jax.experimental import pallas as pl
from jax.experimental.pallas import tpu as pltpu
from jax.experimental.pallas import tpu_sc as plsc

N = 10000
E = 320000
FEAT = 128
NOUT = 40
NC = 2
NS = 16
NW = NC * NS
EPW = E // NS
KE = 112
NCHUNK = 180
EPWP = NCHUNK * KE
NSLOT = 3
NGROUPS = NCHUNK // NSLOT
RPT = 632
NPAD = NS * RPT
DEGW = 128

_f32 = jnp.float32


@functools.lru_cache(maxsize=None)
def _spmm_pair_fn(F):
  mesh = plsc.VectorSubcoreMesh(core_axis_name="c", subcore_axis_name="s")

  scratch = ([pltpu.VMEM_SHARED((NPAD, F), _f32)]
             + [pltpu.VMEM((KE,), jnp.int32) for _ in range(4 * NSLOT)]
             + [pltpu.VMEM((KE, F), _f32) for _ in range(NSLOT)]
             + [pltpu.SemaphoreType.DMA for _ in range(3 * NSLOT)])

  @functools.partial(
      pl.kernel,
      out_type=jax.ShapeDtypeStruct((NC * NPAD, F), _f32),
      mesh=mesh,
      scratch_types=scratch,
  )
  def spmm(x0_hbm, x1_hbm, row0_hbm, col0_hbm, row1_hbm, col1_hbm,
           zeros_hbm, out_hbm, acc, *rest):
    rowbs = [rest[2 * b:2 * b + 2] for b in range(NSLOT)]
    colbs = [rest[2 * NSLOT + 2 * b:2 * NSLOT + 2 * b + 2]
             for b in range(NSLOT)]
    gbufs = rest[4 * NSLOT:5 * NSLOT]
    gsems = rest[5 * NSLOT:6 * NSLOT]
    ssems = rest[6 * NSLOT:7 * NSLOT]
    isems = rest[7 * NSLOT:8 * NSLOT]
    cid = lax.axis_index("c")
    sid = lax.axis_index("s")
    r0 = pl.multiple_of(sid * RPT, 8)
    pltpu.sync_copy(zeros_hbm, acc.at[pl.ds(r0, RPT)])
    plsc.subcore_barrier()

    def pipeline(x_hbm, row_hbm, col_hbm):
      wid = sid

      def fire_idx(c, b, p):
        pltpu.async_copy(row_hbm.at[wid, c], rowbs[b][p], isems[b])
        pltpu.async_copy(col_hbm.at[wid, c], colbs[b][p], isems[b])

      def wait_idx(b, p):
        pltpu.make_async_copy(row_hbm.at[wid, 0], rowbs[b][p],
                              isems[b]).wait()
        pltpu.make_async_copy(col_hbm.at[wid, 0], colbs[b][p],
                              isems[b]).wait()

      def fire_g(b, p):
        pltpu.async_copy(x_hbm.at[colbs[b][p]], gbufs[b], gsems[b])

      def wait_g_fire_s(b, p):
        pltpu.make_async_copy(x_hbm.at[colbs[b][p]], gbufs[b],
                              gsems[b]).wait()
        pltpu.async_copy(gbufs[b], acc.at[rowbs[b][p]], ssems[b], add=True)

      def wait_s(b, p):
        pltpu.make_async_copy(gbufs[b], acc.at[rowbs[b][p]],
                              ssems[b]).wait()

      for b in range(NSLOT):
        fire_idx(b, b, 0)

      def group_pair(m2, carry):
        for p in (0, 1):
          m = 2 * m2 + p
          for b in range(NSLOT):
            c = m * NSLOT + b
            prev = (b + NSLOT - 1) % NSLOT
            if p == 0:
              @pl.when(m >= 1)
              def _():
                wait_s(b, p)
            else:
              wait_s(b, p)
            wait_idx(b, p)
            fire_g(b, p)

            @pl.when(m < NGROUPS - 1)
            def _():
              fire_idx(c + NSLOT, b, 1 - p)

            if b == 0:
              if p == 0:
                @pl.when(m >= 1)
                def _():
                  wait_g_fire_s(prev, 1 - p)
              else:
                wait_g_fire_s(prev, 1 - p)
            else:
              wait_g_fire_s(prev, p)
        return carry

      lax.fori_loop(0, NGROUPS // 2, group_pair, 0)
      wait_g_fire_s(NSLOT - 1, (NGROUPS - 1) % 2)
      for b in range(NSLOT):
        wait_s(b, 0)

    @pl.when(cid == 0)
    def _():
      pipeline(x0_hbm, row0_hbm, col0_hbm)

    @pl.when(cid == 1)
    def _():
      pipeline(x1_hbm, row1_hbm, col1_hbm)

    plsc.subcore_barrier()
    pltpu.sync_copy(acc.at[pl.ds(r0, RPT)],
                    out_hbm.at[pl.ds(pl.multiple_of(cid * NPAD + r0, 8), RPT)])

  return spmm


@functools.lru_cache(maxsize=None)
def _deg_fn():
  mesh = plsc.VectorSubcoreMesh(core_axis_name="c", subcore_axis_name="s")

  scratch = ([
      pltpu.VMEM_SHARED((NPAD, DEGW), _f32),
      pltpu.VMEM((NCHUNK, KE), jnp.int32),
      pltpu.VMEM((KE, DEGW), _f32),
  ] + [pltpu.SemaphoreType.DMA for _ in range(NSLOT)])

  @functools.partial(
      pl.kernel,
      out_type=jax.ShapeDtypeStruct((NC * NPAD, DEGW), _f32),
      mesh=mesh,
      scratch_types=scratch,
  )
  def deg(row0_hbm, row1_hbm, ones_hbm, zeros_hbm, out_hbm, acc, rowb,
          onesb, *ssems):
    cid = lax.axis_index("c")
    sid = lax.axis_index("s")
    r0 = pl.multiple_of(sid * RPT, 8)
    pltpu.sync_copy(zeros_hbm, acc.at[pl.ds(r0, RPT)])

    @pl.when(cid == 0)
    def _():
      pltpu.sync_copy(row0_hbm.at[sid], rowb)

    @pl.when(cid == 1)
    def _():
      pltpu.sync_copy(row1_hbm.at[sid], rowb)

    pltpu.sync_copy(ones_hbm, onesb)
    plsc.subcore_barrier()

    for b in range(NSLOT):
      pltpu.async_copy(onesb, acc.at[rowb.at[b]], ssems[b], add=True)

    def group(m, carry):
      for b in range(NSLOT):
        c = m * NSLOT + b
        pltpu.make_async_copy(onesb, acc.at[rowb.at[c]], ssems[b]).wait()
        pltpu.async_copy(onesb, acc.at[rowb.at[c]], ssems[b], add=True)
      return carry

    lax.fori_loop(1, NGROUPS, group, 0)
    for b in range(NSLOT):
      pltpu.make_async_copy(onesb, acc.at[rowb.at[b]], ssems[b]).wait()
    plsc.subcore_barrier()
    pltpu.sync_copy(acc.at[pl.ds(r0, RPT)],
                    out_hbm.at[pl.ds(pl.multiple_of(cid * NPAD + r0, 8), RPT)])

  return deg


def _deg(row0, row1, onesw, zerosw):
  out = _deg_fn()(row0, row1, onesw, zerosw).reshape(NC, NPAD, DEGW)
  return out[0, :N, 0:1], out[1, :N, 0:1]


def _spmm_pair(x0, x1, row0, col0, row1, col1, zeros128):
  out = _spmm_pair_fn(FEAT)(x0, x1, row0, col0, row1, col1,
                            zeros128).reshape(NC, NPAD, FEAT)
  return out[0, :N], out[1, :N]


def _chunked(idx, fill):
  a = idx.reshape(NS, EPW)
  pad = jnp.full((NS, EPWP - EPW), fill, jnp.int32)
  return jnp.concatenate([a, pad], axis=1).reshape(NS, NCHUNK, KE)


def _dot(a, b):
  return jnp.dot(a, b, preferred_element_type=_f32,
                 precision=lax.Precision.HIGHEST)


def _dinv_of(d):
  return lax.rsqrt(jnp.maximum(d, 1e-6))


def _bn_elu_expr(z, gamma, beta):
  m = jnp.mean(z, axis=0, keepdims=True)
  v = jnp.mean((z - m) ** 2, axis=0, keepdims=True)
  zn = (z - m) * lax.rsqrt(v + 1e-5) * gamma + beta
  return jnp.where(zn > 0, zn, jnp.exp(jnp.minimum(zn, 0.0)) - 1.0)


def _prep_dual(x_e, x_s, deg0, deg1, w_e, w_s):
  def body(xe_ref, xs_ref, d0_ref, d1_ref, we_ref, ws_ref,
           se_ref, ss_ref, ye_ref, ys_ref):
    xe = xe_ref[...]
    xs = xs_ref[...]
    se_ref[...] = _dot(xe, we_ref[...])
    ss_ref[...] = _dot(xs, ws_ref[...])
    ye_ref[...] = _dinv_of(d0_ref[...]) * xe
    ys_ref[...] = _dinv_of(d1_ref[...]) * xs

  sh = jax.ShapeDtypeStruct((N, FEAT), _f32)
  row_spec = pl.BlockSpec((2000, FEAT), lambda i: (i, 0))
  deg_spec = pl.BlockSpec((2000, 1), lambda i: (i, 0))
  w_spec = pl.BlockSpec((FEAT, FEAT), lambda i: (0, 0))
  return pl.pallas_call(
      body, grid=(N // 2000,),
      in_specs=[row_spec, row_spec, deg_spec, deg_spec, w_spec, w_spec],
      out_specs=(row_spec, row_spec, row_spec, row_spec),
      out_shape=(sh, sh, sh, sh))(x_e, x_s, deg0, deg1, w_e, w_s)


def _mid_dual(u_e, u_s, deg0, deg1):
  def body(ue_ref, us_ref, d0_ref, d1_ref, t1e_ref, w2e_ref, t1s_ref,
           w2s_ref):
    dinv0 = _dinv_of(d0_ref[...])
    dinv1 = _dinv_of(d1_ref[...])
    t1e = -dinv0 * ue_ref[...]
    t1s = -dinv1 * us_ref[...]
    t1e_ref[...] = t1e
    w2e_ref[...] = dinv0 * t1e
    t1s_ref[...] = t1s
    w2s_ref[...] = dinv1 * t1s

  sh = jax.ShapeDtypeStruct((N, FEAT), _f32)
  row_spec = pl.BlockSpec((2000, FEAT), lambda i: (i, 0))
  deg_spec = pl.BlockSpec((2000, 1), lambda i: (i, 0))
  return pl.pallas_call(
      body, grid=(N // 2000,),
      in_specs=[row_spec, row_spec, deg_spec, deg_spec],
      out_specs=(row_spec, row_spec, row_spec, row_spec),
      out_shape=(sh, sh, sh, sh))(u_e, u_s, deg0, deg1)


_BLK = 2000


def _conv_pre(g, v, x0, t1, deg, cheb_w, bias, a):
  def body(g_ref, v_ref, x0_ref, t1_ref, d_ref, w_ref, b_ref, a_ref,
           out_ref):
    dinv = _dinv_of(d_ref[...])
    x0_ = x0_ref[...]
    t1_ = t1_ref[...]
    t2 = -2.0 * dinv * v_ref[...] - x0_
    w0 = w_ref[0]
    w1 = w_ref[1]
    w2 = w_ref[2]
    n0 = jnp.sqrt(jnp.sum(w0 * w0))
    n1 = jnp.sqrt(jnp.sum(w1 * w1))
    n2 = jnp.sqrt(jnp.sum(w2 * w2))
    cheb = (_dot(x0_, w0 + 0.01 * n0) + _dot(t1_, w1 + 0.01 * n1)
            + _dot(t2, w2 + 0.01 * n2)) * 0.001
    a_ = a_ref[0, 0]
    out_ref[...] = a_ * g_ref[...] + (1.0 - a_) * cheb + b_ref[...]

  return pl.pallas_call(
      body,
      grid=(N // _BLK,),
      in_specs=[
          pl.BlockSpec((_BLK, FEAT), lambda i: (i, 0)),
          pl.BlockSpec((_BLK, FEAT), lambda i: (i, 0)),
          pl.BlockSpec((_BLK, FEAT), lambda i: (i, 0)),
          pl.BlockSpec((_BLK, FEAT), lambda i: (i, 0)),
          pl.BlockSpec((_BLK, 1), lambda i: (i, 0)),
          pl.BlockSpec((3, FEAT, FEAT), lambda i: (0, 0, 0)),
          pl.BlockSpec((1, FEAT), lambda i: (0, 0)),
          pl.BlockSpec(memory_space=pltpu.SMEM),
      ],
      out_specs=pl.BlockSpec((_BLK, FEAT), lambda i: (i, 0)),
      out_shape=jax.ShapeDtypeStruct((N, FEAT), _f32),
  )(g, v, x0, t1, deg, cheb_w, bias, a)


def _bn_elu(z, gamma, beta):
  def body(z_ref, g_ref, b_ref, out_ref):
    out_ref[...] = _bn_elu_expr(z_ref[...], g_ref[...], b_ref[...])

  return pl.pallas_call(
      body, out_shape=jax.ShapeDtypeStruct((N, FEAT), _f32))(z, gamma, beta)


def _head(h0, h1, w11, b11, g1, be1, w111, b111, g2, be2, w12, b12):
  def body(h0_ref, h1_ref, w11_ref, b11_ref, g1_ref, be1_ref, w111_ref,
           b111_ref, g2_ref, be2_ref, w12_ref, b12_ref, out_ref):
    x1 = jnp.concatenate([h0_ref[...], h1_ref[...]], axis=1)
    z = _dot(x1, w11_ref[...]) + b11_ref[...]
    z = _bn_elu_expr(z, g1_ref[...], be1_ref[...])
    z = _dot(z, w111_ref[...]) + b111_ref[...]
    z = _bn_elu_expr(z, g2_ref[...], be2_ref[...])
    z = _dot(z, w12_ref[...]) + b12_ref[...]
    m = jnp.max(z, axis=1, keepdims=True)
    sz = z - m
    lse = jnp.log(jnp.sum(jnp.exp(sz), axis=1, keepdims=True))
    out_ref[...] = sz - lse

  return pl.pallas_call(
      body, out_shape=jax.ShapeDtypeStruct((N, NOUT), _f32))(
          h0, h1, w11, b11, g1, be1, w111, b111, g2, be2, w12, b12)


def kernel(x, adj0_index, adj1_index, params):
  row0, col0 = _chunked(adj0_index[0], N), _chunked(adj0_index[1], 0)
  row1, col1 = _chunked(adj1_index[0], N), _chunked(adj1_index[1], 0)
  zeros128 = jnp.zeros((RPT, FEAT), _f32)
  zerosw = jnp.zeros((RPT, DEGW), _f32)
  onesw = jnp.ones((KE, DEGW), _f32)

  deg0, deg1 = _deg(row0, row1, onesw, zerosw)

  def pair(a, b):
    return _spmm_pair(a, b, row0, col0, row1, col1, zeros128)

  def dual_conv(x_e, x_s, pe, ps, bne, bns):
    a_e = jax.nn.sigmoid(pe['alpha']).reshape(1, 1)
    a_s = jax.nn.sigmoid(ps['alpha']).reshape(1, 1)
    sup_e, sup_s, y_e, y_s = _prep_dual(x_e, x_s, deg0, deg1,
                                        pe['gcn_weight'], ps['gcn_weight'])
    g_e, g_s = pair(sup_e, sup_s)
    u_e, u_s = pair(y_e, y_s)
    t1_e, w2_e, t1_s, w2_s = _mid_dual(u_e, u_s, deg0, deg1)
    v_e, v_s = pair(w2_e, w2_s)
    pre_e = _conv_pre(g_e, v_e, x_e, t1_e, deg0, pe['cheb_weight'],
                      pe['bias'].reshape(1, FEAT), a_e)
    pre_s = _conv_pre(g_s, v_s, x_s, t1_s, deg1, ps['cheb_weight'],
                      ps['bias'].reshape(1, FEAT), a_s)
    h_e = _bn_elu(pre_e, bne['gamma'].reshape(1, FEAT),
                  bne['beta'].reshape(1, FEAT))
    h_s = _bn_elu(pre_s, bns['gamma'].reshape(1, FEAT),
                  bns['beta'].reshape(1, FEAT))
    return h_e, h_s

  h_e, h_s = dual_conv(x, x, params['gc_in_exp'], params['gc_in_sp'],
                       params['bn_in_exp'], params['bn_in_sp'])
  h_e, h_s = dual_conv(h_e, h_s, params['cgc1_exp'], params['cgc1_sp'],
                       params['bn_c1_exp'], params['bn_c1_sp'])

  return _head(
      h_e, h_s,
      params['W_out11'], params['b_out11'].reshape(1, FEAT),
      params['bn_out1']['gamma'].reshape(1, FEAT),
      params['bn_out1']['beta'].reshape(1, FEAT),
      params['W_out111'], params['b_out111'].reshape(1, FEAT),
      params['bn_out111']['gamma'].reshape(1, FEAT),
      params['bn_out111']['beta'].reshape(1, FEAT),
      params['W_out12'], params['b_out12'].reshape(1, NOUT))
